# baseline (speedup 1.0000x reference)
import functools

import jax
import jax.numpy as jnp
from jax.experimental import pallas as pl
from jax.experimental.pallas import tpu as pltpu

_IN_F = 48
_HID = 32
_OUT_F = 48
_BN_EPS = 1e-5


def _fused_kernel(xt_ref, w1t_ref, w2_ref, w4_ref, aux_ref, b4t_ref,
                  o_ref, h2_buf, s_ref, sq_ref, *, n_total):
    p = pl.program_id(0)
    t = pl.program_id(1)

    @pl.when(p == 0)
    def _stats_phase():
        x = xt_ref[...].astype(jnp.bfloat16)
        w1b = w1t_ref[...].astype(jnp.bfloat16)
        w2b = w2_ref[...].astype(jnp.bfloat16)
        b1t = aux_ref[:, 0:1]
        h1 = jnp.maximum(
            jnp.dot(w1b, x, preferred_element_type=jnp.float32) + b1t,
            0.0)
        h2 = jax.lax.dot_general(
            w2b, h1.astype(jnp.bfloat16), (((0,), (0,)), ((), ())),
            preferred_element_type=jnp.float32)
        s = jnp.sum(h2, axis=1, keepdims=True)
        sq = jnp.sum(h2 * h2, axis=1, keepdims=True)

        @pl.when(t == 0)
        def _():
            s_ref[...] = s
            sq_ref[...] = sq

        @pl.when(t > 0)
        def _():
            s_ref[...] += s
            sq_ref[...] += sq

        h2_buf[t] = h2.astype(jnp.bfloat16)

    @pl.when(p == 1)
    def _apply_phase():
        inv_n = 1.0 / n_total
        mean = s_ref[...] * inv_n
        var = jnp.maximum(sq_ref[...] * inv_n - mean * mean, 0.0)
        inv_std = jax.lax.rsqrt(var + _BN_EPS)
        scale = inv_std * aux_ref[:, 1:2]
        shift = aux_ref[:, 2:3] - mean * scale

        h2 = h2_buf[t].astype(jnp.float32)
        h2 = jnp.maximum(h2 * scale + shift, 0.0)
        w4b = w4_ref[...].astype(jnp.bfloat16)
        out = jax.lax.dot_general(
            w4b, h2.astype(jnp.bfloat16), (((0,), (0,)), ((), ())),
            preferred_element_type=jnp.float32) + b4t_ref[...]
        o_ref[...] = jax.nn.sigmoid(out)


def kernel(x, w1, b1, w2, b2, gamma, beta, w4, b4):
    del b2
    n = x.shape[0]
    tile = 16384
    while n % tile != 0 and tile > 128:
        tile //= 2
    num_tiles = n // tile

    xt = x.T
    w1t = w1.T
    aux = jnp.concatenate([b1.T, gamma.T, beta.T], axis=1)
    b4t = b4.T

    def const(shape):
        return pl.BlockSpec(shape, lambda p, t, _nd=len(shape): (0,) * _nd)

    out_t = pl.pallas_call(
        functools.partial(_fused_kernel, n_total=n),
        out_shape=jax.ShapeDtypeStruct((_OUT_F, n), jnp.float32),
        grid=(2, num_tiles),
        in_specs=[
            pl.BlockSpec((_IN_F, tile),
                         lambda p, t: (0, t * (1 - p) + p * (num_tiles - 1))),
            const((_HID, _IN_F)),
            const((_HID, _HID)),
            const((_HID, _OUT_F)),
            const((_HID, 3)),
            const((_OUT_F, 1)),
        ],
        out_specs=pl.BlockSpec((_OUT_F, tile), lambda p, t: (0, p * t)),
        scratch_shapes=[
            pltpu.VMEM((num_tiles, _HID, tile), jnp.bfloat16),
            pltpu.VMEM((_HID, 1), jnp.float32),
            pltpu.VMEM((_HID, 1), jnp.float32),
        ],
        compiler_params=pltpu.CompilerParams(
            dimension_semantics=("arbitrary", "arbitrary"),
            vmem_limit_bytes=50 * 1024 * 1024,
        ),
    )(xt, w1t, w2, w4, aux, b4t)

    return out_t.T


# speedup vs baseline: 1.1597x; 1.1597x over previous
import functools

import jax
import jax.numpy as jnp
from jax.experimental import pallas as pl
from jax.experimental.pallas import tpu as pltpu

_IN_F = 48
_HID = 32
_OUT_F = 48
_BN_EPS = 1e-5


def _fused_kernel(xt_ref, w1t_ref, w2_ref, w4_ref, aux_ref, b4t_ref,
                  o_ref, h2_buf, s_ref, sq_ref, *, n_total):
    p = pl.program_id(0)
    t = pl.program_id(1)

    @pl.when(p == 0)
    def _stats_phase():
        x = xt_ref[...]
        b1t = aux_ref[:, 0:1]
        h1 = jnp.maximum(
            jnp.dot(w1t_ref[...], x, preferred_element_type=jnp.float32)
            + b1t, 0.0)
        h2 = jax.lax.dot_general(
            w2_ref[...], h1, (((0,), (0,)), ((), ())),
            preferred_element_type=jnp.float32)
        s = jnp.sum(h2, axis=1, keepdims=True)
        sq = jnp.sum(h2 * h2, axis=1, keepdims=True)

        @pl.when(t == 0)
        def _():
            s_ref[...] = s
            sq_ref[...] = sq

        @pl.when(t > 0)
        def _():
            s_ref[...] += s
            sq_ref[...] += sq

        h2_buf[t] = h2.astype(jnp.bfloat16)

    @pl.when(p == 1)
    def _apply_phase():
        inv_n = 1.0 / n_total
        mean = s_ref[...] * inv_n
        var = jnp.maximum(sq_ref[...] * inv_n - mean * mean, 0.0)
        inv_std = jax.lax.rsqrt(var + _BN_EPS)
        scale = inv_std * aux_ref[:, 1:2]
        shift = aux_ref[:, 2:3] - mean * scale

        h2 = h2_buf[t].astype(jnp.float32)
        h2 = jnp.maximum(h2 * scale + shift, 0.0)
        out = jax.lax.dot_general(
            w4_ref[...], h2, (((0,), (0,)), ((), ())),
            preferred_element_type=jnp.float32) + b4t_ref[...]
        o_ref[...] = 0.5 * jnp.tanh(0.5 * out) + 0.5


def kernel(x, w1, b1, w2, b2, gamma, beta, w4, b4):
    del b2
    n = x.shape[0]
    tile = 32768
    while n % tile != 0 and tile > 128:
        tile //= 2
    num_tiles = n // tile

    xt = x.T
    w1t = w1.T
    aux = jnp.concatenate([b1.T, gamma.T, beta.T], axis=1)
    b4t = b4.T

    def const(shape):
        return pl.BlockSpec(shape, lambda p, t, _nd=len(shape): (0,) * _nd)

    out_t = pl.pallas_call(
        functools.partial(_fused_kernel, n_total=n),
        out_shape=jax.ShapeDtypeStruct((_OUT_F, n), jnp.float32),
        grid=(2, num_tiles),
        in_specs=[
            pl.BlockSpec((_IN_F, tile),
                         lambda p, t: (0, t * (1 - p) + p * (num_tiles - 1))),
            const((_HID, _IN_F)),
            const((_HID, _HID)),
            const((_HID, _OUT_F)),
            const((_HID, 3)),
            const((_OUT_F, 1)),
        ],
        out_specs=pl.BlockSpec((_OUT_F, tile), lambda p, t: (0, p * t)),
        scratch_shapes=[
            pltpu.VMEM((num_tiles, _HID, tile), jnp.bfloat16),
            pltpu.VMEM((_HID, 1), jnp.float32),
            pltpu.VMEM((_HID, 1), jnp.float32),
        ],
        compiler_params=pltpu.CompilerParams(
            dimension_semantics=("arbitrary", "arbitrary"),
            vmem_limit_bytes=50 * 1024 * 1024,
        ),
    )(xt, w1t, w2, w4, aux, b4t)

    return out_t.T


# speedup vs baseline: 1.2367x; 1.0664x over previous
import functools

import jax
import jax.numpy as jnp
from jax.experimental import pallas as pl
from jax.experimental.pallas import tpu as pltpu

_IN_F = 48
_HID = 32
_OUT_F = 48
_BN_EPS = 1e-5


def _fused_kernel(xt_ref, w1t_ref, w2_ref, w4_ref, aux_ref, b4t_ref,
                  o_ref, h2_buf, s_ref, sq_ref, *, n_total):
    p = pl.program_id(0)
    t = pl.program_id(1)

    @pl.when(p == 0)
    def _stats_phase():
        x = xt_ref[...]
        b1t = aux_ref[:, 0:1]
        h1 = jnp.maximum(
            jnp.dot(w1t_ref[...], x, preferred_element_type=jnp.float32)
            + b1t, 0.0)
        h2 = jax.lax.dot_general(
            w2_ref[...], h1, (((0,), (0,)), ((), ())),
            preferred_element_type=jnp.float32)
        s = jnp.sum(h2, axis=1, keepdims=True)
        sq = jnp.sum(h2 * h2, axis=1, keepdims=True)

        @pl.when(t == 0)
        def _():
            s_ref[...] = s
            sq_ref[...] = sq

        @pl.when(t > 0)
        def _():
            s_ref[...] += s
            sq_ref[...] += sq

        h2_buf[t] = h2.astype(jnp.bfloat16)

    @pl.when(p == 1)
    def _apply_phase():
        inv_n = 1.0 / n_total
        mean = s_ref[...] * inv_n
        var = jnp.maximum(sq_ref[...] * inv_n - mean * mean, 0.0)
        inv_std = jax.lax.rsqrt(var + _BN_EPS)
        scale = (inv_std * aux_ref[:, 1:2]).astype(jnp.bfloat16)
        shift = (aux_ref[:, 2:3]
                 - mean * inv_std * aux_ref[:, 1:2]).astype(jnp.bfloat16)

        h2 = h2_buf[t]
        h2 = jnp.maximum(h2 * scale + shift, jnp.bfloat16(0.0))
        out = jax.lax.dot_general(
            w4_ref[...].astype(jnp.bfloat16), h2, (((0,), (0,)), ((), ())),
            preferred_element_type=jnp.float32) + b4t_ref[...]
        o_ref[...] = 0.5 * jnp.tanh(0.5 * out) + 0.5


def kernel(x, w1, b1, w2, b2, gamma, beta, w4, b4):
    del b2
    n = x.shape[0]
    tile = 32768
    while n % tile != 0 and tile > 128:
        tile //= 2
    num_tiles = n // tile

    xt = x.T
    w1t = w1.T
    aux = jnp.concatenate([b1.T, gamma.T, beta.T], axis=1)
    b4t = b4.T

    def const(shape):
        return pl.BlockSpec(shape, lambda p, t, _nd=len(shape): (0,) * _nd)

    out_t = pl.pallas_call(
        functools.partial(_fused_kernel, n_total=n),
        out_shape=jax.ShapeDtypeStruct((_OUT_F, n), jnp.float32),
        grid=(2, num_tiles),
        in_specs=[
            pl.BlockSpec((_IN_F, tile),
                         lambda p, t: (0, t * (1 - p) + p * (num_tiles - 1))),
            const((_HID, _IN_F)),
            const((_HID, _HID)),
            const((_HID, _OUT_F)),
            const((_HID, 3)),
            const((_OUT_F, 1)),
        ],
        out_specs=pl.BlockSpec((_OUT_F, tile), lambda p, t: (0, p * t)),
        scratch_shapes=[
            pltpu.VMEM((num_tiles, _HID, tile), jnp.bfloat16),
            pltpu.VMEM((_HID, 1), jnp.float32),
            pltpu.VMEM((_HID, 1), jnp.float32),
        ],
        compiler_params=pltpu.CompilerParams(
            dimension_semantics=("arbitrary", "arbitrary"),
            vmem_limit_bytes=50 * 1024 * 1024,
        ),
    )(xt, w1t, w2, w4, aux, b4t)

    return out_t.T
